# baseline (device time: 82663 ns/iter reference)
import jax
import jax.numpy as jnp
from jax import lax
from jax.experimental import pallas as pl
from jax.experimental.pallas import tpu as pltpu

N_DEV = 4
N_STREAM = 8


def kernel(t):
    m, n = t.shape
    ch = m // N_DEV
    n2 = n // 2
    sr = ch // N_STREAM

    def body(t_hbm, out_hbm, t_vmem, rs_ref, ag_ref,
             in_sems, out_sems, rs_send, rs_recv, ag_send, ag_recv):
        my = lax.axis_index("i")
        left = lax.rem(my + N_DEV - 1, N_DEV)
        right = lax.rem(my + 1, N_DEV)

        barrier_sem = pltpu.get_barrier_semaphore()
        for nbr in (lax.rem(my + N_DEV - 1, N_DEV), lax.rem(my + 1, N_DEV)):
            pl.semaphore_signal(
                barrier_sem, inc=1,
                device_id=(nbr,), device_id_type=pl.DeviceIdType.MESH,
            )

        own_copies = []
        for g in range(N_STREAM):
            cp = pltpu.make_async_copy(
                t_hbm.at[pl.ds(my * ch + g * sr, sr), :],
                t_vmem.at[pl.ds(my * ch + g * sr, sr), :],
                in_sems.at[g],
            )
            cp.start()
            own_copies.append(cp)
        in_copies = []
        for k, off in enumerate((3, 1, 2)):
            c = lax.rem(my + off, N_DEV)
            cp = pltpu.make_async_copy(
                t_hbm.at[pl.ds(c * ch, ch), :],
                t_vmem.at[pl.ds(c * ch, ch), :],
                in_sems.at[N_STREAM + k],
            )
            cp.start()
            in_copies.append(cp)

        dst = (right, left)
        cols = (slice(0, n2), slice(n2, n))

        def rows(c, g):
            return pl.ds(c * ch + g * sr, sr)

        def hop(buf_ref, send_sems, recv_sems, d, g, h):
            return pltpu.make_async_remote_copy(
                src_ref=buf_ref.at[d, g, h],
                dst_ref=buf_ref.at[d, g, h + 1],
                send_sem=send_sems.at[d, g, h],
                recv_sem=recv_sems.at[d, g, h],
                device_id=(dst[d],),
                device_id_type=pl.DeviceIdType.MESH,
            )

        rs_rdmas = []
        live = {}
        own_copies[0].wait()
        for d in range(2):
            rs_ref[d, 0, 0, :, :] = (
                t_vmem[rows(my, 0), cols[d]].astype(jnp.bfloat16)
            )
        pl.semaphore_wait(barrier_sem, 2)
        for g in range(N_STREAM):
            if g > 0:
                own_copies[g].wait()
                for d in range(2):
                    rs_ref[d, g, 0, :, :] = (
                        t_vmem[rows(my, g), cols[d]].astype(jnp.bfloat16)
                    )
            for d in range(2):
                r = hop(rs_ref, rs_send, rs_recv, d, g, 0)
                r.start()
                rs_rdmas.append(r)
                live[(d, g)] = r
        for cp in in_copies:
            cp.wait()

        def f_of(s_bf16):
            s = s_bf16.astype(jnp.float32)
            r = jnp.maximum(s, 0.0)
            return (jnp.tanh(s) * s * s + r * r * r).astype(jnp.bfloat16)

        out_copies = []

        def emit(d, g, slot, chunk_id):
            cp = pltpu.make_async_copy(
                ag_ref.at[d, g, slot],
                out_hbm.at[rows(chunk_id, g), cols[d]],
                out_sems.at[d, g, slot],
            )
            cp.start()
            out_copies.append(cp)

        ag_rdmas = []
        own = (lax.rem(my + 1, N_DEV), lax.rem(my + N_DEV - 1, N_DEV))
        for h in range(N_DEV - 1):
            for g in range(N_STREAM):
                for d in range(2):
                    live[(d, g)].wait_recv()
                    c = lax.rem(my + N_DEV - 1 - h, N_DEV) if d == 0 else (
                        lax.rem(my + 1 + h, N_DEV)
                    )
                    rs_ref[d, g, h + 1, :, :] = (
                        rs_ref[d, g, h + 1, :, :]
                        + t_vmem[rows(c, g), cols[d]].astype(jnp.bfloat16)
                    )
                    if h + 1 < N_DEV - 1:
                        r = hop(rs_ref, rs_send, rs_recv, d, g, h + 1)
                        r.start()
                        rs_rdmas.append(r)
                        live[(d, g)] = r
                    else:
                        ag_ref[d, g, 0, :, :] = f_of(rs_ref[d, g, N_DEV - 1, :, :])
                        r = hop(ag_ref, ag_send, ag_recv, d, g, 0)
                        r.start()
                        ag_rdmas.append(r)
                        live[(d, g)] = r
                        emit(d, g, 0, own[d])

        for r in rs_rdmas:
            r.wait_send()

        for h in range(N_DEV - 1):
            for g in range(N_STREAM):
                for d in range(2):
                    live[(d, g)].wait_recv()
                    if h + 1 < N_DEV - 1:
                        r = hop(ag_ref, ag_send, ag_recv, d, g, h + 1)
                        r.start()
                        ag_rdmas.append(r)
                        live[(d, g)] = r
                    oc = lax.rem(my + N_DEV - h, N_DEV) if d == 0 else (
                        lax.rem(my + h, N_DEV)
                    )
                    emit(d, g, h + 1, oc)

        for r in ag_rdmas:
            r.wait_send()
        for cp in out_copies:
            cp.wait()

    return pl.pallas_call(
        body,
        out_shape=jax.ShapeDtypeStruct((m, n), jnp.bfloat16),
        in_specs=[pl.BlockSpec(memory_space=pl.ANY)],
        out_specs=pl.BlockSpec(memory_space=pltpu.MemorySpace.HBM),
        scratch_shapes=[
            pltpu.VMEM((m, n), jnp.float32),
            pltpu.VMEM((2, N_STREAM, N_DEV, sr, n2), jnp.bfloat16),
            pltpu.VMEM((2, N_STREAM, N_DEV, sr, n2), jnp.bfloat16),
            pltpu.SemaphoreType.DMA((N_STREAM + N_DEV - 1,)),
            pltpu.SemaphoreType.DMA((2, N_STREAM, N_DEV)),
            pltpu.SemaphoreType.DMA((2, N_STREAM, N_DEV - 1)),
            pltpu.SemaphoreType.DMA((2, N_STREAM, N_DEV - 1)),
            pltpu.SemaphoreType.DMA((2, N_STREAM, N_DEV - 1)),
            pltpu.SemaphoreType.DMA((2, N_STREAM, N_DEV - 1)),
        ],
        compiler_params=pltpu.CompilerParams(
            collective_id=0,
            vmem_limit_bytes=100 * 1024 * 1024,
        ),
    )(t)


# device time: 81697 ns/iter; 1.0118x vs baseline; 1.0118x over previous
import jax
import jax.numpy as jnp
from jax import lax
from jax.experimental import pallas as pl
from jax.experimental.pallas import tpu as pltpu

N_DEV = 4
N_STREAM = 4


def kernel(t):
    m, n = t.shape
    ch = m // N_DEV
    n2 = n // 2
    sr = ch // N_STREAM

    def body(t_hbm, out_hbm, t_vmem, rs_ref, ag_ref,
             in_sems, out_sems, rs_send, rs_recv, ag_send, ag_recv):
        my = lax.axis_index("i")
        left = lax.rem(my + N_DEV - 1, N_DEV)
        right = lax.rem(my + 1, N_DEV)

        barrier_sem = pltpu.get_barrier_semaphore()
        for nbr in (lax.rem(my + N_DEV - 1, N_DEV), lax.rem(my + 1, N_DEV)):
            pl.semaphore_signal(
                barrier_sem, inc=1,
                device_id=(nbr,), device_id_type=pl.DeviceIdType.MESH,
            )

        own_copies = []
        for g in range(N_STREAM):
            cp = pltpu.make_async_copy(
                t_hbm.at[pl.ds(my * ch + g * sr, sr), :],
                t_vmem.at[pl.ds(my * ch + g * sr, sr), :],
                in_sems.at[g],
            )
            cp.start()
            own_copies.append(cp)
        in_copies = []
        for k, off in enumerate((3, 1, 2)):
            c = lax.rem(my + off, N_DEV)
            cp = pltpu.make_async_copy(
                t_hbm.at[pl.ds(c * ch, ch), :],
                t_vmem.at[pl.ds(c * ch, ch), :],
                in_sems.at[N_STREAM + k],
            )
            cp.start()
            in_copies.append(cp)

        dst = (right, left)
        cols = (slice(0, n2), slice(n2, n))

        def rows(c, g):
            return pl.ds(c * ch + g * sr, sr)

        def hop(buf_ref, send_sems, recv_sems, d, g, h):
            return pltpu.make_async_remote_copy(
                src_ref=buf_ref.at[d, g, h],
                dst_ref=buf_ref.at[d, g, h + 1],
                send_sem=send_sems.at[d, g, h],
                recv_sem=recv_sems.at[d, g, h],
                device_id=(dst[d],),
                device_id_type=pl.DeviceIdType.MESH,
            )

        rs_rdmas = []
        live = {}
        own_copies[0].wait()
        for d in range(2):
            rs_ref[d, 0, 0, :, :] = (
                t_vmem[rows(my, 0), cols[d]].astype(jnp.bfloat16)
            )
        pl.semaphore_wait(barrier_sem, 2)
        for g in range(N_STREAM):
            if g > 0:
                own_copies[g].wait()
                for d in range(2):
                    rs_ref[d, g, 0, :, :] = (
                        t_vmem[rows(my, g), cols[d]].astype(jnp.bfloat16)
                    )
            for d in range(2):
                r = hop(rs_ref, rs_send, rs_recv, d, g, 0)
                r.start()
                rs_rdmas.append(r)
                live[(d, g)] = r
        for cp in in_copies:
            cp.wait()

        def f_of(s_bf16):
            s = s_bf16.astype(jnp.float32)
            r = jnp.maximum(s, 0.0)
            return (jnp.tanh(s) * s * s + r * r * r).astype(jnp.bfloat16)

        out_copies = []

        def emit(d, g, slot, chunk_id):
            cp = pltpu.make_async_copy(
                ag_ref.at[d, g, slot],
                out_hbm.at[rows(chunk_id, g), cols[d]],
                out_sems.at[d, g, slot],
            )
            cp.start()
            out_copies.append(cp)

        ag_rdmas = []
        own = (lax.rem(my + 1, N_DEV), lax.rem(my + N_DEV - 1, N_DEV))
        for h in range(N_DEV - 1):
            for g in range(N_STREAM):
                for d in range(2):
                    live[(d, g)].wait_recv()
                    c = lax.rem(my + N_DEV - 1 - h, N_DEV) if d == 0 else (
                        lax.rem(my + 1 + h, N_DEV)
                    )
                    rs_ref[d, g, h + 1, :, :] = (
                        rs_ref[d, g, h + 1, :, :]
                        + t_vmem[rows(c, g), cols[d]].astype(jnp.bfloat16)
                    )
                    if h + 1 < N_DEV - 1:
                        r = hop(rs_ref, rs_send, rs_recv, d, g, h + 1)
                        r.start()
                        rs_rdmas.append(r)
                        live[(d, g)] = r
                    else:
                        ag_ref[d, g, 0, :, :] = f_of(rs_ref[d, g, N_DEV - 1, :, :])
                        r = hop(ag_ref, ag_send, ag_recv, d, g, 0)
                        r.start()
                        ag_rdmas.append(r)
                        live[(d, g)] = r
                        emit(d, g, 0, own[d])

        for r in rs_rdmas:
            r.wait_send()

        for h in range(N_DEV - 1):
            for g in range(N_STREAM):
                for d in range(2):
                    live[(d, g)].wait_recv()
                    if h + 1 < N_DEV - 1:
                        r = hop(ag_ref, ag_send, ag_recv, d, g, h + 1)
                        r.start()
                        ag_rdmas.append(r)
                        live[(d, g)] = r
                    oc = lax.rem(my + N_DEV - h, N_DEV) if d == 0 else (
                        lax.rem(my + h, N_DEV)
                    )
                    emit(d, g, h + 1, oc)

        for r in ag_rdmas:
            r.wait_send()
        for cp in out_copies:
            cp.wait()

    return pl.pallas_call(
        body,
        out_shape=jax.ShapeDtypeStruct((m, n), jnp.bfloat16),
        in_specs=[pl.BlockSpec(memory_space=pl.ANY)],
        out_specs=pl.BlockSpec(memory_space=pltpu.MemorySpace.HBM),
        scratch_shapes=[
            pltpu.VMEM((m, n), jnp.float32),
            pltpu.VMEM((2, N_STREAM, N_DEV, sr, n2), jnp.bfloat16),
            pltpu.VMEM((2, N_STREAM, N_DEV, sr, n2), jnp.bfloat16),
            pltpu.SemaphoreType.DMA((N_STREAM + N_DEV - 1,)),
            pltpu.SemaphoreType.DMA((2, N_STREAM, N_DEV)),
            pltpu.SemaphoreType.DMA((2, N_STREAM, N_DEV - 1)),
            pltpu.SemaphoreType.DMA((2, N_STREAM, N_DEV - 1)),
            pltpu.SemaphoreType.DMA((2, N_STREAM, N_DEV - 1)),
            pltpu.SemaphoreType.DMA((2, N_STREAM, N_DEV - 1)),
        ],
        compiler_params=pltpu.CompilerParams(
            collective_id=0,
            vmem_limit_bytes=100 * 1024 * 1024,
        ),
    )(t)
